# baseline (speedup 1.0000x reference)
import math
from functools import partial

import jax
import jax.numpy as jnp
from jax.experimental import pallas as pl
from jax.experimental.pallas import tpu as pltpu

_ERF_ALPHA = (
    -2.72614225801306e-10, 2.77068142495902e-08, -2.10102402082508e-06,
    -5.69250639462346e-05, -7.34990630326855e-04, -2.95459980854025e-03,
    -1.60960333262415e-02,
)
_ERF_BETA = (
    -1.45660718464996e-05, -2.13374055278905e-04, -1.68282697438203e-03,
    -7.37332916720468e-03, -1.42647390514189e-02,
)


def _gelu(x):
    z = jnp.clip(x * 0.7071067811865476, -4.0, 4.0)
    z2 = z * z
    p = jnp.full_like(z2, _ERF_ALPHA[0])
    for c in _ERF_ALPHA[1:]:
        p = p * z2 + c
    q = jnp.full_like(z2, _ERF_BETA[0])
    for c in _ERF_BETA[1:]:
        q = q * z2 + c
    return 0.5 * x * (1.0 + z * p / q)


def _encoder_kernel(x_ref, wqkv_ref, wo_ref, w1_ref, w2_ref, vec_ref, o_ref,
                    *, nhead, bt, seq):
    f32 = jnp.float32
    bf16 = jnp.bfloat16
    D, N = x_ref.shape
    F = w1_ref.shape[0]
    hd = D // nhead

    x = x_ref[...]

    vec = vec_ref[...]
    b_qkv = vec[0:3 * D, 0:1]
    b_out = vec[0:D, 1:2]
    b1 = vec[0:F, 2:3]
    b2 = vec[0:D, 3:4]
    g1 = vec[0:D, 4:5]
    be1 = vec[0:D, 5:6]
    g2 = vec[0:D, 6:7]
    be2 = vec[0:D, 7:8]

    def ln(z, g, b):
        mu = jnp.mean(z, axis=0, keepdims=True)
        zc = z - mu
        var = jnp.mean(zc * zc, axis=0, keepdims=True)
        return zc * jax.lax.rsqrt(var + 1e-5) * g + b

    y = ln(x, g1, be1).astype(bf16)
    qkv = jnp.dot(wqkv_ref[...], y, preferred_element_type=f32) + b_qkv
    qkv = qkv.astype(bf16)

    ctx_rows = []
    for h in range(nhead):
        r = h * hd
        per_b = []
        for b in range(bt):
            c0 = b * seq
            qh = qkv[r:r + hd, c0:c0 + seq]
            kh = qkv[D + r:D + r + hd, c0:c0 + seq]
            vh = qkv[2 * D + r:2 * D + r + hd, c0:c0 + seq]
            s = jax.lax.dot_general(qh, kh, (((0,), (0,)), ((), ())),
                                    preferred_element_type=f32)
            s = s - jnp.max(s, axis=1, keepdims=True)
            p = jnp.exp(s)
            p = (p * pl.reciprocal(jnp.sum(p, axis=1, keepdims=True),
                                   approx=True)).astype(bf16)
            c = jax.lax.dot_general(vh, p, (((1,), (1,)), ((), ())),
                                    preferred_element_type=f32)
            per_b.append(c.astype(bf16))
        ctx_rows.append(jnp.concatenate(per_b, axis=1) if bt > 1 else per_b[0])
    ctxT = jnp.concatenate(ctx_rows, axis=0)

    attn = jnp.dot(wo_ref[...], ctxT, preferred_element_type=f32) + b_out
    x1 = x + attn

    y2 = ln(x1, g2, be2).astype(bf16)
    h1 = jnp.dot(w1_ref[...], y2, preferred_element_type=f32) + b1
    h1 = _gelu(h1).astype(bf16)
    o_ref[...] = x1 + jnp.dot(w2_ref[...], h1, preferred_element_type=f32) + b2


def kernel(src, w_in, b_in, w_out, b_out, w1, b1, w2, b2, g1, be1, g2, be2):
    S, B, D = src.shape
    H = 12
    hd = D // H
    F = w1.shape[0]
    scale = 1.0 / math.sqrt(hd)
    f32, bf16 = jnp.float32, jnp.bfloat16

    BT = 2
    grid = (B // BT,)
    N = BT * S

    xT = src.astype(f32).transpose(2, 1, 0).reshape(D, B * S)

    qscale = jnp.concatenate([jnp.full((D,), scale, f32),
                              jnp.ones((2 * D,), f32)])
    wqkv = (w_in * qscale[:, None]).astype(bf16)
    b_qkv = (b_in * qscale).astype(f32)
    wob = w_out.astype(bf16)
    w1b = w1.astype(bf16)
    w2b = w2.astype(bf16)

    cols = [b_qkv, b_out, b1, b2, g1, be1, g2, be2]
    R = max(F, 3 * D)
    vec = jnp.zeros((R, 8), f32)
    for i, c in enumerate(cols):
        vec = vec.at[:c.shape[0], i].set(c.astype(f32))

    def _resident(shape):
        nd = len(shape)
        return pl.BlockSpec(shape, lambda b, _nd=nd: (0,) * _nd)

    out = pl.pallas_call(
        partial(_encoder_kernel, nhead=H, bt=BT, seq=S),
        out_shape=jax.ShapeDtypeStruct((D, B * S), f32),
        grid=grid,
        in_specs=[pl.BlockSpec((D, N), lambda b: (0, b)),
                  _resident(wqkv.shape), _resident(wob.shape),
                  _resident(w1b.shape), _resident(w2b.shape),
                  _resident(vec.shape)],
        out_specs=pl.BlockSpec((D, N), lambda b: (0, b)),
        compiler_params=pltpu.CompilerParams(
            dimension_semantics=("parallel",),
            vmem_limit_bytes=64 * 1024 * 1024,
        ),
    )(xT, wqkv, wob, w1b, w2b, vec)

    return out.reshape(D, B, S).transpose(2, 1, 0)


# speedup vs baseline: 1.9548x; 1.9548x over previous
import math
from functools import partial

import jax
import jax.numpy as jnp
from jax.experimental import pallas as pl
from jax.experimental.pallas import tpu as pltpu


def _gelu_tanh(x):
    u = 0.7978845608028654 * (x + 0.044715 * (x * x * x))
    return 0.5 * x * (1.0 + jnp.tanh(u))


def _encoder_kernel(x_ref, wqkv_ref, wo_ref, w1_ref, w2_ref, vec_ref, o_ref,
                    *, nhead, bt, seq, scale):
    f32 = jnp.float32
    bf16 = jnp.bfloat16
    S, BT, D = x_ref.shape
    N = BT * S
    F = w1_ref.shape[0]
    hd = D // nhead

    xT = jnp.concatenate([x_ref[:, b, :].T for b in range(bt)], axis=1)

    vec = vec_ref[...]
    b_qkv = vec[0:3 * D, 0:1]
    b_out = vec[0:D, 1:2]
    b1 = vec[0:F, 2:3]
    b2 = vec[0:D, 3:4]
    g1 = vec[0:D, 4:5]
    be1 = vec[0:D, 5:6]
    g2 = vec[0:D, 6:7]
    be2 = vec[0:D, 7:8]

    def ln(z, g, b):
        mu = jnp.mean(z, axis=0, keepdims=True)
        zc = z - mu
        var = jnp.mean(zc * zc, axis=0, keepdims=True)
        return zc * jax.lax.rsqrt(var + 1e-5) * g + b

    y = ln(xT, g1, be1).astype(bf16)
    qkv = (jnp.dot(wqkv_ref[...], y, preferred_element_type=f32)
           + b_qkv).astype(bf16)

    qkv3 = qkv.reshape(3 * nhead, hd, N)
    q3 = qkv3[0:nhead]
    k3 = qkv3[nhead:2 * nhead]
    v3 = qkv3[2 * nhead:3 * nhead]
    ctx_cols = []
    for b in range(bt):
        c0 = b * seq
        qb = q3[:, :, c0:c0 + seq]
        kb = k3[:, :, c0:c0 + seq]
        vb = v3[:, :, c0:c0 + seq]
        s = jnp.einsum('heq,hek->hqk', qb, kb,
                       preferred_element_type=f32)
        s = s * scale
        s = s - jnp.max(s, axis=2, keepdims=True)
        p = jnp.exp(s)
        p = (p * pl.reciprocal(jnp.sum(p, axis=2, keepdims=True),
                               approx=True)).astype(bf16)
        c = jnp.einsum('hek,hqk->heq', vb, p,
                       preferred_element_type=f32)
        ctx_cols.append(c.reshape(D, seq).astype(bf16))
    ctxT = jnp.concatenate(ctx_cols, axis=1)

    attn = jnp.dot(wo_ref[...], ctxT, preferred_element_type=f32) + b_out
    x1 = xT + attn

    y2 = ln(x1, g2, be2).astype(bf16)
    h1 = jnp.dot(w1_ref[...], y2, preferred_element_type=f32) + b1
    h1 = _gelu_tanh(h1).astype(bf16)
    out = x1 + jnp.dot(w2_ref[...], h1, preferred_element_type=f32) + b2

    for b in range(bt):
        o_ref[:, b, :] = out[:, b * seq:(b + 1) * seq].T


def kernel(src, w_in, b_in, w_out, b_out, w1, b1, w2, b2, g1, be1, g2, be2):
    S, B, D = src.shape
    H = 12
    hd = D // H
    F = w1.shape[0]
    scale = 1.0 / math.sqrt(hd)
    f32, bf16 = jnp.float32, jnp.bfloat16

    BT = 8 if B % 8 == 0 else B
    grid = (B // BT,)

    wqkv = w_in.astype(bf16)
    wob = w_out.astype(bf16)
    w1b = w1.astype(bf16)
    w2b = w2.astype(bf16)

    cols = [b_in, b_out, b1, b2, g1, be1, g2, be2]
    R = max(F, 3 * D)
    vec = jnp.zeros((R, 8), f32)
    for i, c in enumerate(cols):
        vec = vec.at[:c.shape[0], i].set(c.astype(f32))

    def _resident(shape):
        nd = len(shape)
        return pl.BlockSpec(shape, lambda b, _nd=nd: (0,) * _nd)

    out = pl.pallas_call(
        partial(_encoder_kernel, nhead=H, bt=BT, seq=S, scale=scale),
        out_shape=jax.ShapeDtypeStruct((S, B, D), f32),
        grid=grid,
        in_specs=[pl.BlockSpec((S, BT, D), lambda b: (0, b, 0)),
                  _resident(wqkv.shape), _resident(wob.shape),
                  _resident(w1b.shape), _resident(w2b.shape),
                  _resident(vec.shape)],
        out_specs=pl.BlockSpec((S, BT, D), lambda b: (0, b, 0)),
        compiler_params=pltpu.CompilerParams(
            dimension_semantics=("parallel",),
            vmem_limit_bytes=64 * 1024 * 1024,
        ),
    )(src.astype(f32), wqkv, wob, w1b, w2b, vec)

    return out


# speedup vs baseline: 2.8815x; 1.4741x over previous
import math
from functools import partial

import jax
import jax.numpy as jnp
from jax.experimental import pallas as pl
from jax.experimental.pallas import tpu as pltpu


def _gelu_tanh(x):
    u = 0.7978845608028654 * (x + 0.044715 * (x * x * x))
    return 0.5 * x * (1.0 + jnp.tanh(u))


def _encoder_kernel(x_ref, win_hbm, wout_hbm, w1_hbm, w2_hbm,
                    bqkv_r, bout_r, b1_r, b2_r, g1_r, be1_r, g2_r, be2_r,
                    o_ref,
                    wqkv_b, wo_b, w1_b, w2_b, stage_a, stage_b,
                    sem_a, sem_b,
                    *, nhead, bt, seq, scale):
    f32 = jnp.float32
    bf16 = jnp.bfloat16
    S, BT, D = x_ref.shape
    N = BT * S
    F = w1_b.shape[0]
    hd = D // nhead
    half = F // 2
    w2c = D // 4

    def start_a(hbm, r0, slot):
        pltpu.make_async_copy(hbm.at[pl.ds(r0, D), :], stage_a.at[slot],
                              sem_a.at[slot]).start()

    def take_a(hbm, slot, dst, r0):
        pltpu.make_async_copy(hbm.at[pl.ds(0, D), :], stage_a.at[slot],
                              sem_a.at[slot]).wait()
        dst[pl.ds(r0, D), :] = stage_a[slot].astype(bf16)

    def start_b(r0, slot):
        pltpu.make_async_copy(w2_hbm.at[pl.ds(r0, w2c), :], stage_b.at[slot],
                              sem_b.at[slot]).start()

    def take_b(slot, r0):
        pltpu.make_async_copy(w2_hbm.at[pl.ds(0, w2c), :], stage_b.at[slot],
                              sem_b.at[slot]).wait()
        w2_b[pl.ds(r0, w2c), :] = stage_b[slot].astype(bf16)

    start_a(win_hbm, 0, 0)
    start_a(win_hbm, D, 1)

    xT = jnp.concatenate([x_ref[:, b, :].T for b in range(bt)], axis=1)

    b_qkv = bqkv_r[...].reshape(3 * D, 1)
    b_out = bout_r[...].reshape(D, 1)
    b1 = b1_r[...].reshape(F, 1)
    b2 = b2_r[...].reshape(D, 1)
    g1 = g1_r[...].reshape(D, 1)
    be1 = be1_r[...].reshape(D, 1)
    g2 = g2_r[...].reshape(D, 1)
    be2 = be2_r[...].reshape(D, 1)

    def ln(z, g, b):
        mu = jnp.mean(z, axis=0, keepdims=True)
        zc = z - mu
        var = jnp.mean(zc * zc, axis=0, keepdims=True)
        return zc * jax.lax.rsqrt(var + 1e-5) * g + b

    y = ln(xT, g1, be1).astype(bf16)

    take_a(win_hbm, 0, wqkv_b, 0)
    start_a(win_hbm, 2 * D, 0)
    take_a(win_hbm, 1, wqkv_b, D)
    start_a(wout_hbm, 0, 1)
    take_a(win_hbm, 0, wqkv_b, 2 * D)
    start_a(w1_hbm, 0, 0)

    qkv = (jnp.dot(wqkv_b[...], y, preferred_element_type=f32)
           + b_qkv).astype(bf16)

    take_a(wout_hbm, 1, wo_b, 0)
    start_a(w1_hbm, D, 1)

    qkv3 = qkv.reshape(3 * nhead, hd, N)
    q3 = qkv3[0:nhead]
    k3 = qkv3[nhead:2 * nhead]
    v3 = qkv3[2 * nhead:3 * nhead]

    def attend(b):
        c0 = b * seq
        qb = q3[:, :, c0:c0 + seq]
        kb = k3[:, :, c0:c0 + seq]
        vb = v3[:, :, c0:c0 + seq]
        s = jnp.einsum('heq,hek->hqk', qb, kb,
                       preferred_element_type=f32)
        s = s * scale
        s = s - jnp.max(s, axis=2, keepdims=True)
        p = jnp.exp(s)
        p = (p * pl.reciprocal(jnp.sum(p, axis=2, keepdims=True),
                               approx=True)).astype(bf16)
        c = jnp.einsum('hek,hqk->heq', vb, p,
                       preferred_element_type=f32)
        return c.reshape(D, seq).astype(bf16)

    ctx_cols = [attend(b) for b in range(bt // 2)]

    take_a(w1_hbm, 0, w1_b, 0)
    start_a(w1_hbm, 2 * D, 0)
    take_a(w1_hbm, 1, w1_b, D)
    start_a(w1_hbm, 3 * D, 1)

    ctx_cols += [attend(b) for b in range(bt // 2, bt)]
    ctxT = jnp.concatenate(ctx_cols, axis=1)

    take_a(w1_hbm, 0, w1_b, 2 * D)
    start_b(0, 0)
    take_a(w1_hbm, 1, w1_b, 3 * D)
    start_b(w2c, 1)

    attn = jnp.dot(wo_b[...], ctxT, preferred_element_type=f32) + b_out
    x1 = xT + attn
    y2 = ln(x1, g2, be2).astype(bf16)

    take_b(0, 0)
    start_b(2 * w2c, 0)
    take_b(1, w2c)
    start_b(3 * w2c, 1)

    h1a = jnp.dot(w1_b[0:half, :], y2, preferred_element_type=f32) + b1[0:half]
    h1a = _gelu_tanh(h1a).astype(bf16)

    take_b(0, 2 * w2c)
    take_b(1, 3 * w2c)

    h1b = jnp.dot(w1_b[half:F, :], y2, preferred_element_type=f32) + b1[half:F]
    h1b = _gelu_tanh(h1b).astype(bf16)

    out = (x1 + b2
           + jnp.dot(w2_b[:, 0:half], h1a, preferred_element_type=f32)
           + jnp.dot(w2_b[:, half:F], h1b, preferred_element_type=f32))

    for b in range(bt):
        o_ref[:, b, :] = out[:, b * seq:(b + 1) * seq].T


def kernel(src, w_in, b_in, w_out, b_out, w1, b1, w2, b2, g1, be1, g2, be2):
    S, B, D = src.shape
    H = 12
    hd = D // H
    F = w1.shape[0]
    scale = 1.0 / math.sqrt(hd)
    f32, bf16 = jnp.float32, jnp.bfloat16

    BT = 8 if B % 8 == 0 else B
    grid = (B // BT,)

    def _row(v):
        return v.reshape(1, v.shape[0]).astype(f32)

    def _vmem(shape):
        nd = len(shape)
        return pl.BlockSpec(shape, lambda b, _nd=nd: (0,) * _nd)

    hbm = pl.BlockSpec(memory_space=pl.ANY)

    out = pl.pallas_call(
        partial(_encoder_kernel, nhead=H, bt=BT, seq=S, scale=scale),
        out_shape=jax.ShapeDtypeStruct((S, B, D), f32),
        grid=grid,
        in_specs=[pl.BlockSpec((S, BT, D), lambda b: (0, b, 0)),
                  hbm, hbm, hbm, hbm,
                  _vmem((1, 3 * D)), _vmem((1, D)), _vmem((1, F)),
                  _vmem((1, D)), _vmem((1, D)), _vmem((1, D)),
                  _vmem((1, D)), _vmem((1, D))],
        out_specs=pl.BlockSpec((S, BT, D), lambda b: (0, b, 0)),
        scratch_shapes=[pltpu.VMEM((3 * D, D), bf16),
                        pltpu.VMEM((D, D), bf16),
                        pltpu.VMEM((F, D), bf16),
                        pltpu.VMEM((D, F), bf16),
                        pltpu.VMEM((2, D, D), f32),
                        pltpu.VMEM((2, D // 4, F), f32),
                        pltpu.SemaphoreType.DMA((2,)),
                        pltpu.SemaphoreType.DMA((2,))],
        compiler_params=pltpu.CompilerParams(
            dimension_semantics=("parallel",),
            vmem_limit_bytes=64 * 1024 * 1024,
        ),
    )(src.astype(f32), w_in.astype(f32), w_out.astype(f32),
      w1.astype(f32), w2.astype(f32),
      _row(b_in), _row(b_out), _row(b1), _row(b2),
      _row(g1), _row(be1), _row(g2), _row(be2))

    return out
